# baseline (speedup 1.0000x reference)
import functools

import jax
import jax.numpy as jnp
from jax import lax
from jax.experimental import pallas as pl
from jax.experimental.pallas import tpu as pltpu
from jax.experimental.pallas import tpu_sc as plsc

N = 10000
NP = 10240
E = 320000
IN_CH = 128
HID = 128
DH = 64
D2 = 16
NC = 2
NS = 16
K = 128
KC = 160
KCH = KC // 2
CAP = NS * KC * K
RPT = NP // NS


def _mesh():
    return plsc.VectorSubcoreMesh(core_axis_name="c", subcore_axis_name="s")


def _deg(dsts16, ones16, zeros16):
    @functools.partial(
        pl.kernel,
        out_type=jax.ShapeDtypeStruct((NC, NP, D2), jnp.float32),
        mesh=_mesh(),
        scratch_types=[
            pltpu.VMEM((KCH, K), jnp.int32),
            pltpu.VMEM((K, D2), jnp.float32),
            pltpu.VMEM_SHARED((NP, D2), jnp.float32),
        ],
        compiler_params=pltpu.CompilerParams(use_tc_tiling_on_sc=False),
    )
    def deg_kernel(dsts_hbm, ones_hbm, zeros_hbm, out_hbm, dst_v, ones_v, acc_sh):
        cid = lax.axis_index("c")
        sid = lax.axis_index("s")
        pltpu.sync_copy(dsts_hbm.at[sid, pl.ds(cid * KCH, KCH)], dst_v)
        pltpu.sync_copy(ones_hbm, ones_v)
        r0 = sid * RPT
        pltpu.sync_copy(zeros_hbm.at[pl.ds(r0, RPT)], acc_sh.at[pl.ds(r0, RPT)])
        plsc.subcore_barrier()

        @pl.loop(0, KCH)
        def _(g):
            pltpu.sync_copy(ones_v, acc_sh.at[dst_v.at[g]], add=True)

        plsc.subcore_barrier()
        pltpu.sync_copy(acc_sh.at[pl.ds(r0, RPT)], out_hbm.at[cid, pl.ds(r0, RPT)])

    return deg_kernel(dsts16, ones16, zeros16)


def _agg_body(y_hbm, zeros_hbm, out_hbm, src_v, dst_v, buf_a, buf_b, acc_sh,
              sem_a, sem_b, kc):
    r0 = lax.axis_index("s") * RPT
    pltpu.sync_copy(zeros_hbm.at[pl.ds(r0, RPT)], acc_sh.at[pl.ds(r0, RPT)])
    plsc.subcore_barrier()

    pltpu.async_copy(y_hbm.at[src_v.at[0]], buf_a, sem_a)

    @pl.loop(0, kc, step=2)
    def _(g):
        pltpu.async_copy(y_hbm.at[src_v.at[g + 1]], buf_b, sem_b)
        pltpu.make_async_copy(y_hbm.at[src_v.at[g]], buf_a, sem_a).wait()
        pltpu.sync_copy(buf_a, acc_sh.at[dst_v.at[g]], add=True)

        @pl.when(g + 2 < kc)
        def _():
            pltpu.async_copy(y_hbm.at[src_v.at[g + 2]], buf_a, sem_a)

        pltpu.make_async_copy(y_hbm.at[src_v.at[g + 1]], buf_b, sem_b).wait()
        pltpu.sync_copy(buf_b, acc_sh.at[dst_v.at[g + 1]], add=True)

    plsc.subcore_barrier()
    cid = lax.axis_index("c")
    pltpu.sync_copy(acc_sh.at[pl.ds(r0, RPT)], out_hbm.at[cid, pl.ds(r0, RPT)])


def _agg1(ystack, srcs16s, dsts16, zeros64):
    @functools.partial(
        pl.kernel,
        out_type=jax.ShapeDtypeStruct((NC, NP, DH), jnp.float32),
        mesh=_mesh(),
        scratch_types=[
            pltpu.VMEM((KC, K), jnp.int32),
            pltpu.VMEM((KC, K), jnp.int32),
            pltpu.VMEM((K, DH), jnp.float32),
            pltpu.VMEM((K, DH), jnp.float32),
            pltpu.VMEM_SHARED((NP, DH), jnp.float32),
            pltpu.SemaphoreType.DMA,
            pltpu.SemaphoreType.DMA,
        ],
        compiler_params=pltpu.CompilerParams(use_tc_tiling_on_sc=False),
    )
    def agg1_kernel(y_hbm, srcs_hbm, dsts_hbm, zeros_hbm, out_hbm,
                    src_v, dst_v, buf_a, buf_b, acc_sh, sem_a, sem_b):
        cid = lax.axis_index("c")
        sid = lax.axis_index("s")
        pltpu.sync_copy(srcs_hbm.at[cid, sid], src_v)
        pltpu.sync_copy(dsts_hbm.at[sid], dst_v)
        _agg_body(y_hbm, zeros_hbm, out_hbm, src_v, dst_v,
                  buf_a, buf_b, acc_sh, sem_a, sem_b, KC)

    return agg1_kernel(ystack, srcs16s, dsts16, zeros64)


def _agg2(y2, srcs16, dsts16, zeros16):
    @functools.partial(
        pl.kernel,
        out_type=jax.ShapeDtypeStruct((NC, NP, D2), jnp.float32),
        mesh=_mesh(),
        scratch_types=[
            pltpu.VMEM((KCH, K), jnp.int32),
            pltpu.VMEM((KCH, K), jnp.int32),
            pltpu.VMEM((K, D2), jnp.float32),
            pltpu.VMEM((K, D2), jnp.float32),
            pltpu.VMEM_SHARED((NP, D2), jnp.float32),
            pltpu.SemaphoreType.DMA,
            pltpu.SemaphoreType.DMA,
        ],
        compiler_params=pltpu.CompilerParams(use_tc_tiling_on_sc=False),
    )
    def agg2_kernel(y_hbm, srcs_hbm, dsts_hbm, zeros_hbm, out_hbm,
                    src_v, dst_v, buf_a, buf_b, acc_sh, sem_a, sem_b):
        cid = lax.axis_index("c")
        sid = lax.axis_index("s")
        pltpu.sync_copy(srcs_hbm.at[sid, pl.ds(cid * KCH, KCH)], src_v)
        pltpu.sync_copy(dsts_hbm.at[sid, pl.ds(cid * KCH, KCH)], dst_v)
        _agg_body(y_hbm, zeros_hbm, out_hbm, src_v, dst_v,
                  buf_a, buf_b, acc_sh, sem_a, sem_b, KCH)

    return agg2_kernel(y2, srcs16, dsts16, zeros16)


def _prep1(x_pad, W1s, degs):
    G = 8
    BR = NP // G

    def body(x_ref, w_ref, dg_ref, y_ref):
        dga = dg_ref[0, :, 0:1]
        dgb = dg_ref[1, :, 0:1]
        dinv = lax.rsqrt(dga + dgb + 1.0)
        xw = jnp.dot(x_ref[...], w_ref[0], preferred_element_type=jnp.float32)
        y_ref[0, :, :] = dinv * xw

    return pl.pallas_call(
        body,
        grid=(NC, G),
        in_specs=[
            pl.BlockSpec((BR, IN_CH), lambda c, i: (i, 0)),
            pl.BlockSpec((1, IN_CH, DH), lambda c, i: (c, 0, 0)),
            pl.BlockSpec((NC, BR, D2), lambda c, i: (0, i, 0)),
        ],
        out_specs=pl.BlockSpec((1, BR, DH), lambda c, i: (c, i, 0)),
        out_shape=jax.ShapeDtypeStruct((NC, NP, DH), jnp.float32),
    )(x_pad, W1s, degs)


def _prep2(acc1, y1s, degs, b1r, W2p):
    G = 8
    BR = NP // G

    def body(a0_ref, a1_ref, y0_ref, y1_ref, dg_ref, b1_ref, w2_ref, y2_ref):
        dinv = lax.rsqrt(dg_ref[0, :, 0:1] + dg_ref[1, :, 0:1] + 1.0)
        pre = jnp.concatenate(
            [a0_ref[0] + y0_ref[0], a1_ref[0] + y1_ref[0]], axis=1)
        h1 = jnp.maximum(dinv * pre + b1_ref[...], 0.0)
        y2_ref[...] = dinv * jnp.dot(h1, w2_ref[...], preferred_element_type=jnp.float32)

    half0 = pl.BlockSpec((1, BR, DH), lambda i: (0, i, 0))
    half1 = pl.BlockSpec((1, BR, DH), lambda i: (1, i, 0))
    return pl.pallas_call(
        body,
        grid=(G,),
        in_specs=[
            half0, half1, half0, half1,
            pl.BlockSpec((NC, BR, D2), lambda i: (0, i, 0)),
            pl.BlockSpec((1, HID), lambda i: (0, 0)),
            pl.BlockSpec((HID, D2), lambda i: (0, 0)),
        ],
        out_specs=pl.BlockSpec((BR, D2), lambda i: (i, 0)),
        out_shape=jax.ShapeDtypeStruct((NP, D2), jnp.float32),
    )(acc1, acc1, y1s, y1s, degs, b1r, W2p)


def _head(acc2, y2, degs, b2p, wlp, blp):
    def body(aa_ref, ab_ref, y2_ref, dg_ref, b2_ref, wl_ref, bl_ref, out_ref):
        dinv = lax.rsqrt(dg_ref[0, :, 0:1] + dg_ref[1, :, 0:1] + 1.0)
        h2 = jnp.maximum(
            dinv * (aa_ref[0] + ab_ref[0] + y2_ref[...]) + b2_ref[...], 0.0)
        rows = lax.broadcasted_iota(jnp.int32, (NP, 1), 0)
        gmean = jnp.sum(jnp.where(rows < N, h2, 0.0), axis=0) * (1.0 / N)
        res = jnp.sum(gmean[None, :] * wl_ref[...], axis=1) + bl_ref[0, :]
        out_ref[...] = res[None, :]

    return pl.pallas_call(
        body,
        grid=(1,),
        in_specs=[
            pl.BlockSpec((1, NP, D2), lambda i: (0, 0, 0)),
            pl.BlockSpec((1, NP, D2), lambda i: (1, 0, 0)),
            pl.BlockSpec((NP, D2), lambda i: (0, 0)),
            pl.BlockSpec((NC, NP, D2), lambda i: (0, 0, 0)),
            pl.BlockSpec((1, D2), lambda i: (0, 0)),
            pl.BlockSpec((D2, D2), lambda i: (0, 0)),
            pl.BlockSpec((1, D2), lambda i: (0, 0)),
        ],
        out_specs=pl.BlockSpec((1, D2), lambda i: (0, 0)),
        out_shape=jax.ShapeDtypeStruct((1, D2), jnp.float32),
    )(acc2, acc2, y2, degs, b2p, wlp, blp)


def kernel(x, edge_index, W1, b1, W2, b2, Wl, bl):
    src = edge_index[0].astype(jnp.int32)
    dst = edge_index[1].astype(jnp.int32)

    pad16 = jnp.full((CAP - E,), N, jnp.int32)
    srcs16 = jnp.concatenate([src, pad16]).reshape(NS, KC, K)
    dsts16 = jnp.concatenate([dst, pad16]).reshape(NS, KC, K)
    srcs16s = jnp.stack([srcs16, srcs16 + NP])

    x_pad = jnp.zeros((NP, IN_CH), jnp.float32).at[:N].set(x)
    W1s = jnp.stack([W1[:, :DH], W1[:, DH:]])
    zeros64 = jnp.zeros((NP, DH), jnp.float32)
    zeros16 = jnp.zeros((NP, D2), jnp.float32)
    ones16 = jnp.ones((K, D2), jnp.float32)
    W2p = jnp.zeros((HID, D2), jnp.float32).at[:, :2].set(W2)
    b1r = b1.reshape(1, HID)
    b2p = jnp.zeros((1, D2), jnp.float32).at[0, :2].set(b2)
    wlp = jnp.zeros((D2, D2), jnp.float32).at[:2, :2].set(Wl)
    blp = jnp.zeros((1, D2), jnp.float32).at[0, :2].set(bl)

    degs = _deg(dsts16, ones16, zeros16)
    y1s = _prep1(x_pad, W1s, degs)
    ystack = y1s.reshape(NC * NP, DH)
    acc1 = _agg1(ystack, srcs16s, dsts16, zeros64)
    y2 = _prep2(acc1, y1s, degs, b1r, W2p)
    acc2 = _agg2(y2, srcs16, dsts16, zeros16)
    res = _head(acc2, y2, degs, b2p, wlp, blp)
    return res[0, :2]


# speedup vs baseline: 1.3806x; 1.3806x over previous
import functools

import jax
import jax.numpy as jnp
from jax import lax
from jax.experimental import pallas as pl
from jax.experimental.pallas import tpu as pltpu
from jax.experimental.pallas import tpu_sc as plsc

N = 10000
NP = 10240
E = 320000
IN_CH = 128
HID = 128
DH = 64
D2 = 16
NC = 2
NS = 16
NW = NC * NS
K = 128
KC32 = 80
KC16 = 158
CAP32 = NW * KC32 * K
CAP16 = NS * KC16 * K
RPT = NP // NS


def _mesh():
    return plsc.VectorSubcoreMesh(core_axis_name="c", subcore_axis_name="s")


def _deg(dsts32, ones16, zeros16):
    @functools.partial(
        pl.kernel,
        out_type=jax.ShapeDtypeStruct((NC, NP, D2), jnp.float32),
        mesh=_mesh(),
        scratch_types=[
            pltpu.VMEM((KC32, K), jnp.int32),
            pltpu.VMEM((K, D2), jnp.float32),
            pltpu.VMEM_SHARED((NP, D2), jnp.float32),
        ],
        compiler_params=pltpu.CompilerParams(use_tc_tiling_on_sc=False),
    )
    def deg_kernel(dsts_hbm, ones_hbm, zeros_hbm, out_hbm, dst_v, ones_v, acc_sh):
        cid = lax.axis_index("c")
        sid = lax.axis_index("s")
        w = cid * NS + sid
        pltpu.sync_copy(dsts_hbm.at[w], dst_v)
        pltpu.sync_copy(ones_hbm, ones_v)
        r0 = sid * RPT
        pltpu.sync_copy(zeros_hbm.at[pl.ds(r0, RPT)], acc_sh.at[pl.ds(r0, RPT)])
        plsc.subcore_barrier()

        @pl.loop(0, KC32)
        def _(g):
            pltpu.sync_copy(ones_v, acc_sh.at[dst_v.at[g]], add=True)

        plsc.subcore_barrier()
        pltpu.sync_copy(acc_sh.at[pl.ds(r0, RPT)], out_hbm.at[cid, pl.ds(r0, RPT)])

    return deg_kernel(dsts32, ones16, zeros16)


def _agg_body(y_hbm, srcs_hbm, dsts_hbm, zeros_hbm, out_hbm,
              src_v, dst_v, buf_a, buf_b, acc_sh, sem_a, sem_b, kc):
    r0 = lax.axis_index("s") * RPT
    pltpu.sync_copy(zeros_hbm.at[pl.ds(r0, RPT)], acc_sh.at[pl.ds(r0, RPT)])
    plsc.subcore_barrier()

    pltpu.async_copy(y_hbm.at[src_v.at[0]], buf_a, sem_a)

    @pl.loop(0, kc, step=2)
    def _(g):
        pltpu.async_copy(y_hbm.at[src_v.at[g + 1]], buf_b, sem_b)
        pltpu.make_async_copy(y_hbm.at[src_v.at[g]], buf_a, sem_a).wait()
        pltpu.sync_copy(buf_a, acc_sh.at[dst_v.at[g]], add=True)

        @pl.when(g + 2 < kc)
        def _():
            pltpu.async_copy(y_hbm.at[src_v.at[g + 2]], buf_a, sem_a)

        pltpu.make_async_copy(y_hbm.at[src_v.at[g + 1]], buf_b, sem_b).wait()
        pltpu.sync_copy(buf_b, acc_sh.at[dst_v.at[g + 1]], add=True)

    plsc.subcore_barrier()
    cid = lax.axis_index("c")
    pltpu.sync_copy(acc_sh.at[pl.ds(r0, RPT)], out_hbm.at[cid, pl.ds(r0, RPT)])


def _agg1(ystack, srcs16s, dsts16, zeros64):
    @functools.partial(
        pl.kernel,
        out_type=jax.ShapeDtypeStruct((NC, NP, DH), jnp.float32),
        mesh=_mesh(),
        scratch_types=[
            pltpu.VMEM((KC16, K), jnp.int32),
            pltpu.VMEM((KC16, K), jnp.int32),
            pltpu.VMEM((K, DH), jnp.float32),
            pltpu.VMEM((K, DH), jnp.float32),
            pltpu.VMEM_SHARED((NP, DH), jnp.float32),
            pltpu.SemaphoreType.DMA,
            pltpu.SemaphoreType.DMA,
        ],
        compiler_params=pltpu.CompilerParams(use_tc_tiling_on_sc=False),
    )
    def agg1_kernel(y_hbm, srcs_hbm, dsts_hbm, zeros_hbm, out_hbm,
                    src_v, dst_v, buf_a, buf_b, acc_sh, sem_a, sem_b):
        cid = lax.axis_index("c")
        sid = lax.axis_index("s")
        pltpu.sync_copy(srcs_hbm.at[cid, sid], src_v)
        pltpu.sync_copy(dsts_hbm.at[sid], dst_v)
        _agg_body(y_hbm, srcs_hbm, dsts_hbm, zeros_hbm, out_hbm,
                  src_v, dst_v, buf_a, buf_b, acc_sh, sem_a, sem_b, KC16)

    return agg1_kernel(ystack, srcs16s, dsts16, zeros64)


def _agg2(y2, srcs32, dsts32, zeros16):
    @functools.partial(
        pl.kernel,
        out_type=jax.ShapeDtypeStruct((NC, NP, D2), jnp.float32),
        mesh=_mesh(),
        scratch_types=[
            pltpu.VMEM((KC32, K), jnp.int32),
            pltpu.VMEM((KC32, K), jnp.int32),
            pltpu.VMEM((K, D2), jnp.float32),
            pltpu.VMEM((K, D2), jnp.float32),
            pltpu.VMEM_SHARED((NP, D2), jnp.float32),
            pltpu.SemaphoreType.DMA,
            pltpu.SemaphoreType.DMA,
        ],
        compiler_params=pltpu.CompilerParams(use_tc_tiling_on_sc=False),
    )
    def agg2_kernel(y_hbm, srcs_hbm, dsts_hbm, zeros_hbm, out_hbm,
                    src_v, dst_v, buf_a, buf_b, acc_sh, sem_a, sem_b):
        cid = lax.axis_index("c")
        sid = lax.axis_index("s")
        w = cid * NS + sid
        pltpu.sync_copy(srcs_hbm.at[w], src_v)
        pltpu.sync_copy(dsts_hbm.at[w], dst_v)
        _agg_body(y_hbm, srcs_hbm, dsts_hbm, zeros_hbm, out_hbm,
                  src_v, dst_v, buf_a, buf_b, acc_sh, sem_a, sem_b, KC32)

    return agg2_kernel(y2, srcs32, dsts32, zeros16)


def _prep1(x_pad, W1s, degs):
    G = 10
    BR = NP // G

    def body(x_ref, w_ref, dg_ref, y_ref):
        dinv = lax.rsqrt(dg_ref[0, :, 0:1] + dg_ref[1, :, 0:1] + 1.0)
        xw = jnp.dot(x_ref[...], w_ref[0], preferred_element_type=jnp.float32)
        y_ref[0, :, :] = dinv * xw

    return pl.pallas_call(
        body,
        grid=(NC, G),
        in_specs=[
            pl.BlockSpec((BR, IN_CH), lambda c, i: (i, 0)),
            pl.BlockSpec((1, IN_CH, DH), lambda c, i: (c, 0, 0)),
            pl.BlockSpec((NC, BR, D2), lambda c, i: (0, i, 0)),
        ],
        out_specs=pl.BlockSpec((1, BR, DH), lambda c, i: (c, i, 0)),
        out_shape=jax.ShapeDtypeStruct((NC, NP, DH), jnp.float32),
    )(x_pad, W1s, degs)


def _prep2(acc1, y1s, degs, b1r, W2p):
    G = 10
    BR = NP // G

    def body(a0_ref, a1_ref, y0_ref, y1_ref, dg_ref, b1_ref, w2_ref, y2_ref):
        dinv = lax.rsqrt(dg_ref[0, :, 0:1] + dg_ref[1, :, 0:1] + 1.0)
        pre = jnp.concatenate(
            [a0_ref[0] + y0_ref[0], a1_ref[0] + y1_ref[0]], axis=1)
        h1 = jnp.maximum(dinv * pre + b1_ref[...], 0.0)
        y2_ref[...] = dinv * jnp.dot(h1, w2_ref[...], preferred_element_type=jnp.float32)

    half0 = pl.BlockSpec((1, BR, DH), lambda i: (0, i, 0))
    half1 = pl.BlockSpec((1, BR, DH), lambda i: (1, i, 0))
    return pl.pallas_call(
        body,
        grid=(G,),
        in_specs=[
            half0, half1, half0, half1,
            pl.BlockSpec((NC, BR, D2), lambda i: (0, i, 0)),
            pl.BlockSpec((1, HID), lambda i: (0, 0)),
            pl.BlockSpec((HID, D2), lambda i: (0, 0)),
        ],
        out_specs=pl.BlockSpec((BR, D2), lambda i: (i, 0)),
        out_shape=jax.ShapeDtypeStruct((NP, D2), jnp.float32),
    )(acc1, acc1, y1s, y1s, degs, b1r, W2p)


def _head(acc2, y2, degs, b2p, wlp, blp):
    def body(aa_ref, ab_ref, y2_ref, dg_ref, b2_ref, wl_ref, bl_ref, out_ref):
        dinv = lax.rsqrt(dg_ref[0, :, 0:1] + dg_ref[1, :, 0:1] + 1.0)
        h2 = jnp.maximum(
            dinv * (aa_ref[0] + ab_ref[0] + y2_ref[...]) + b2_ref[...], 0.0)
        rows = lax.broadcasted_iota(jnp.int32, (NP, 1), 0)
        gmean = jnp.sum(jnp.where(rows < N, h2, 0.0), axis=0) * (1.0 / N)
        res = jnp.sum(gmean[None, :] * wl_ref[...], axis=1) + bl_ref[0, :]
        out_ref[...] = res[None, :]

    return pl.pallas_call(
        body,
        grid=(1,),
        in_specs=[
            pl.BlockSpec((1, NP, D2), lambda i: (0, 0, 0)),
            pl.BlockSpec((1, NP, D2), lambda i: (1, 0, 0)),
            pl.BlockSpec((NP, D2), lambda i: (0, 0)),
            pl.BlockSpec((NC, NP, D2), lambda i: (0, 0, 0)),
            pl.BlockSpec((1, D2), lambda i: (0, 0)),
            pl.BlockSpec((D2, D2), lambda i: (0, 0)),
            pl.BlockSpec((1, D2), lambda i: (0, 0)),
        ],
        out_specs=pl.BlockSpec((1, D2), lambda i: (0, 0)),
        out_shape=jax.ShapeDtypeStruct((1, D2), jnp.float32),
    )(acc2, acc2, y2, degs, b2p, wlp, blp)


def kernel(x, edge_index, W1, b1, W2, b2, Wl, bl):
    src = edge_index[0].astype(jnp.int32)
    dst = edge_index[1].astype(jnp.int32)

    pad32 = jnp.full((CAP32 - E,), N, jnp.int32)
    srcs32 = jnp.concatenate([src, pad32]).reshape(NW, KC32, K)
    dsts32 = jnp.concatenate([dst, pad32]).reshape(NW, KC32, K)

    pad16 = jnp.full((CAP16 - E,), N, jnp.int32)
    srcs16 = jnp.concatenate([src, pad16]).reshape(NS, KC16, K)
    dsts16 = jnp.concatenate([dst, pad16]).reshape(NS, KC16, K)
    srcs16s = jnp.stack([srcs16, srcs16 + NP])

    x_pad = jnp.zeros((NP, IN_CH), jnp.float32).at[:N].set(x)
    W1s = jnp.stack([W1[:, :DH], W1[:, DH:]])
    zeros64 = jnp.zeros((NP, DH), jnp.float32)
    zeros16 = jnp.zeros((NP, D2), jnp.float32)
    ones16 = jnp.ones((K, D2), jnp.float32)
    W2p = jnp.zeros((HID, D2), jnp.float32).at[:, :2].set(W2)
    b1r = b1.reshape(1, HID)
    b2p = jnp.zeros((1, D2), jnp.float32).at[0, :2].set(b2)
    wlp = jnp.zeros((D2, D2), jnp.float32).at[:2, :2].set(Wl)
    blp = jnp.zeros((1, D2), jnp.float32).at[0, :2].set(bl)

    degs = _deg(dsts32, ones16, zeros16)
    y1s = _prep1(x_pad, W1s, degs)
    ystack = y1s.reshape(NC * NP, DH)
    acc1 = _agg1(ystack, srcs16s, dsts16, zeros64)
    y2 = _prep2(acc1, y1s, degs, b1r, W2p)
    acc2 = _agg2(y2, srcs32, dsts32, zeros16)
    res = _head(acc2, y2, degs, b2p, wlp, blp)
    return res[0, :2]
